# Initial kernel scaffold; baseline (speedup 1.0000x reference)
import jax
import jax.numpy as jnp
from jax.experimental import pallas as pl


def kernel(rpn_cls_scores, rpn_bbox_preds, roi_cls_scores, roi_bbox_preds, anchors, gt_bboxes, gt_labels, rois):
    raise NotImplementedError("write your pallas kernel here")


# speedup vs baseline: 10.3877x; 10.3877x over previous
import functools

import jax
import jax.numpy as jnp
from jax.experimental import pallas as pl
from jax.experimental.pallas import tpu as pltpu

_N = 22500
_NPAD = 22528
_NROWS = 176
_R = 2000
_RPAD = 2048
_RROWS = 16
_G = 32
_C = 21
_BIG = 1 << 30


def _iou_terms(ax1, ay1, ax2, ay2, area_a, gx1, gy1, gx2, gy2):
    area_b = (gx2 - gx1) * (gy2 - gy1)
    ix = jnp.minimum(ax2, gx2) - jnp.maximum(ax1, gx1)
    iy = jnp.minimum(ay2, gy2) - jnp.maximum(ay1, gy1)
    inter = jnp.maximum(ix, 0.0) * jnp.maximum(iy, 0.0)
    return inter / (area_a + area_b - inter + 1e-9)


def _bbox2loc_planes(sx1, sy1, sx2, sy2, dx1, dy1, dx2, dy2):
    w = jnp.maximum(sx2 - sx1, 1e-3)
    h = jnp.maximum(sy2 - sy1, 1e-3)
    cx = sx1 + 0.5 * w
    cy = sy1 + 0.5 * h
    gw = jnp.maximum(dx2 - dx1, 1e-3)
    gh = jnp.maximum(dy2 - dy1, 1e-3)
    gcx = dx1 + 0.5 * gw
    gcy = dy1 + 0.5 * gh
    tx = (gcx - cx) / w
    ty = (gcy - cy) / h
    tw = jnp.log(gw / w)
    th = jnp.log(gh / h)
    return tx, ty, tw, th


def _smooth_l1_sum(pred, tgt, pos):
    d = pos * (pred - tgt)
    ad = jnp.abs(d)
    l = jnp.where(ad < 1.0, 0.5 * ad * ad, ad - 0.5)
    return jnp.sum(l * pos)


def _body(cls_ref, box_ref, anc_ref, roi_ref, rcls_ref, rbox_ref,
          gtb_ref, gtl_ref, o_ref):
    f32 = jnp.float32

    ax1, ay1, ax2, ay2 = anc_ref[0], anc_ref[1], anc_ref[2], anc_ref[3]
    area_a = (ax2 - ax1) * (ay2 - ay1)
    lin = (jax.lax.broadcasted_iota(jnp.int32, (_NROWS, 128), 0) * 128
           + jax.lax.broadcasted_iota(jnp.int32, (_NROWS, 128), 1))
    valid = lin < _N

    run_max = jnp.full((_NROWS, 128), -1.0, f32)
    sgx1 = jnp.zeros((_NROWS, 128), f32)
    sgy1 = jnp.zeros((_NROWS, 128), f32)
    sgx2 = jnp.zeros((_NROWS, 128), f32)
    sgy2 = jnp.zeros((_NROWS, 128), f32)
    best = jnp.zeros((_NROWS, 128), jnp.bool_)
    for g in range(_G):
        gx1 = gtb_ref[g, 0]
        gy1 = gtb_ref[g, 1]
        gx2 = gtb_ref[g, 2]
        gy2 = gtb_ref[g, 3]
        iou = _iou_terms(ax1, ay1, ax2, ay2, area_a, gx1, gy1, gx2, gy2)
        upd = iou > run_max
        run_max = jnp.where(upd, iou, run_max)
        sgx1 = jnp.where(upd, gx1, sgx1)
        sgy1 = jnp.where(upd, gy1, sgy1)
        sgx2 = jnp.where(upd, gx2, sgx2)
        sgy2 = jnp.where(upd, gy2, sgy2)
        m = jnp.max(iou)
        mi = jnp.min(jnp.where(iou == m, lin, _BIG))
        best = jnp.logical_or(best, lin == mi)

    lab = jnp.full((_NROWS, 128), -1, jnp.int32)
    lab = jnp.where(run_max < 0.3, 0, lab)
    lab = jnp.where(run_max >= 0.7, 1, lab)
    lab = jnp.where(best, 1, lab)
    lab = jnp.where(valid, lab, -1)

    tx, ty, tw, th = _bbox2loc_planes(ax1, ay1, ax2, ay2,
                                      sgx1, sgy1, sgx2, sgy2)
    pos = (lab == 1).astype(f32)
    acc = _smooth_l1_sum(box_ref[0], tx, pos)
    acc += _smooth_l1_sum(box_ref[1], ty, pos)
    acc += _smooth_l1_sum(box_ref[2], tw, pos)
    acc += _smooth_l1_sum(box_ref[3], th, pos)
    rpn_loc_loss = acc / jnp.maximum(jnp.sum(pos), 1.0)

    c0, c1 = cls_ref[0], cls_ref[1]
    mm = jnp.maximum(c0, c1)
    lse = mm + jnp.log(jnp.exp(c0 - mm) + jnp.exp(c1 - mm))
    vmask = (lab >= 0).astype(f32)
    nll = lse - jnp.where(lab == 1, c1, c0)
    rpn_cls_loss = (jnp.sum(nll * vmask)
                    / jnp.maximum(jnp.sum(vmask), 1.0))

    rx1, ry1, rx2, ry2 = roi_ref[0], roi_ref[1], roi_ref[2], roi_ref[3]
    rarea = (rx2 - rx1) * (ry2 - ry1)
    rlin = (jax.lax.broadcasted_iota(jnp.int32, (_RROWS, 128), 0) * 128
            + jax.lax.broadcasted_iota(jnp.int32, (_RROWS, 128), 1))
    rvalid = rlin < _R

    rmax = jnp.full((_RROWS, 128), -1.0, f32)
    hgx1 = jnp.zeros((_RROWS, 128), f32)
    hgy1 = jnp.zeros((_RROWS, 128), f32)
    hgx2 = jnp.zeros((_RROWS, 128), f32)
    hgy2 = jnp.zeros((_RROWS, 128), f32)
    slab = jnp.zeros((_RROWS, 128), jnp.int32)
    for g in range(_G):
        gx1 = gtb_ref[g, 0]
        gy1 = gtb_ref[g, 1]
        gx2 = gtb_ref[g, 2]
        gy2 = gtb_ref[g, 3]
        iou = _iou_terms(rx1, ry1, rx2, ry2, rarea, gx1, gy1, gx2, gy2)
        upd = iou > rmax
        rmax = jnp.where(upd, iou, rmax)
        hgx1 = jnp.where(upd, gx1, hgx1)
        hgy1 = jnp.where(upd, gy1, hgy1)
        hgx2 = jnp.where(upd, gx2, hgx2)
        hgy2 = jnp.where(upd, gy2, hgy2)
        slab = jnp.where(upd, gtl_ref[g], slab)

    pos_r = jnp.logical_and(rmax >= 0.5, rvalid)
    rlab = jnp.where(pos_r, slab, 0)
    rlab = jnp.where(rvalid, rlab, -1)

    cm = rcls_ref[0]
    for c in range(1, _C):
        cm = jnp.maximum(cm, rcls_ref[c])
    es = jnp.zeros((_RROWS, 128), f32)
    pk = jnp.zeros((_RROWS, 128), f32)
    for c in range(_C):
        plane = rcls_ref[c]
        es += jnp.exp(plane - cm)
        pk += jnp.where(rlab == c, plane, 0.0)
    rlse = cm + jnp.log(es)
    rv = (rlab >= 0).astype(f32)
    roi_cls_loss = (jnp.sum((rlse - pk) * rv)
                    / jnp.maximum(jnp.sum(rv), 1.0))

    posf = pos_r.astype(f32)
    ttx, tty, ttw, tth = _bbox2loc_planes(rx1, ry1, rx2, ry2,
                                          hgx1, hgy1, hgx2, hgy2)
    racc = jnp.zeros((), f32)
    for j, tgt in enumerate((ttx, tty, ttw, tth)):
        sel = jnp.zeros((_RROWS, 128), f32)
        for c in range(_C):
            sel += jnp.where(rlab == c, rbox_ref[c * 4 + j], 0.0)
        racc += _smooth_l1_sum(sel, tgt, posf)
    roi_loc_loss = racc / jnp.maximum(jnp.sum(posf), 1.0)

    o_ref[0] = rpn_loc_loss
    o_ref[1] = rpn_cls_loss
    o_ref[2] = roi_loc_loss
    o_ref[3] = roi_cls_loss


def _to_planes(x, npad, rows):
    xp = jnp.pad(x, ((0, npad - x.shape[0]), (0, 0)))
    return xp.T.reshape(x.shape[1], rows, 128)


@jax.jit
def kernel(rpn_cls_scores, rpn_bbox_preds, roi_cls_scores, roi_bbox_preds,
           anchors, gt_bboxes, gt_labels, rois):
    cls2d = jnp.transpose(rpn_cls_scores, (0, 2, 3, 1)).reshape(-1, 2)
    box2d = jnp.transpose(rpn_bbox_preds, (0, 2, 3, 1)).reshape(-1, 4)
    clsP = _to_planes(cls2d, _NPAD, _NROWS)
    boxP = _to_planes(box2d, _NPAD, _NROWS)
    ancP = _to_planes(anchors, _NPAD, _NROWS)
    roiP = _to_planes(rois[:, 1:], _RPAD, _RROWS)
    rclsP = _to_planes(roi_cls_scores, _RPAD, _RROWS)
    rboxP = _to_planes(roi_bbox_preds, _RPAD, _RROWS)

    vspec = pl.BlockSpec(memory_space=pltpu.VMEM)
    sspec = pl.BlockSpec(memory_space=pltpu.SMEM)
    out = pl.pallas_call(
        _body,
        out_shape=jax.ShapeDtypeStruct((4,), jnp.float32),
        in_specs=[vspec, vspec, vspec, vspec, vspec, vspec, sspec, sspec],
        out_specs=sspec,
    )(clsP, boxP, ancP, roiP, rclsP, rboxP, gt_bboxes, gt_labels)
    return out[0], out[1], out[2], out[3]
